# baseline (speedup 1.0000x reference)
import functools

import jax
import jax.numpy as jnp
from jax import lax
from jax.experimental import pallas as pl
from jax.experimental.pallas import tpu as pltpu
from jax.experimental.pallas import tpu_sc as plsc

_D = 64
_SEQ = 200
_LANES = 16
_PAD = 64

_NC = 2
_NS = 16
_NW = _NC * _NS

_W = 128
_DB = _D // 8
_STRIDE = _D + 1
_TSTRIDE = _W + 1


def _emb_body(batch, ids_hbm, tok_hbm, pos_hbm, out_hbm, idx_all, rows_v,
              tr_v, pos_v, gsem0, gsem1, ssem0, ssem1):
    wid = lax.axis_index("s") * _NC + lax.axis_index("c")
    b0 = wid * _W

    pltpu.sync_copy(ids_hbm.at[:, pl.ds(b0, _W)], idx_all)
    pltpu.sync_copy(pos_hbm, pos_v)

    lane = lax.iota(jnp.int32, _LANES)
    gsems = (gsem0, gsem1)
    ssems = (ssem0, ssem1)

    def fire_gather(s, q):
        pltpu.async_copy(tok_hbm.at[idx_all.at[s]], rows_v.at[q], gsems[q])

    def wait_gather(s, q):
        pltpu.make_async_copy(tok_hbm.at[idx_all.at[s]], rows_v.at[q],
                              gsems[q]).wait()

    def tr_view(p):
        return tr_v.at[p, :, :, pl.ds(0, _W)]

    def fire_store(s, p):
        pltpu.async_copy(tr_view(p), out_hbm.at[s, :, wid, :, :], ssems[p])

    def wait_store(s, p):
        pltpu.make_async_copy(tr_view(p), out_hbm.at[s, :, wid, :, :],
                              ssems[p]).wait()

    def stage(s, p):
        q = 1 - p

        @pl.when(s + 1 < _SEQ)
        def _():
            fire_gather(s + 1, q)

        wait_gather(s, p)

        @pl.when(s >= 2)
        def _():
            wait_store(s - 2, p)

        posk = [pos_v[s, pl.ds(k * _LANES, _LANES)]
                for k in range(_D // _LANES)]
        dbv, d8v = [], []
        for k in range(_D // _LANES):
            d = k * _LANES + lane
            dbv.append(d >> 3)
            d8v.append(d & 7)

        def blk_body(b, _):
            bvec = jnp.full((_LANES,), b, jnp.int32)
            for k in range(_D // _LANES):
                v = rows_v[p, b, pl.ds(k * _LANES, _LANES)] + posk[k]
                plsc.store_scatter(tr_v.at[p], [dbv[k], d8v[k], bvec], v)
            return 0

        lax.fori_loop(0, _W, blk_body, 0, unroll=8)

        fire_store(s, p)

    fire_gather(0, 0)

    def loop_body(i, _):
        stage(2 * i, 0)
        stage(2 * i + 1, 1)
        return 0

    lax.fori_loop(0, _SEQ // 2, loop_body, 0)

    wait_store(_SEQ - 2, 0)
    wait_store(_SEQ - 1, 1)


def _make_lookup(batch):
    mesh = plsc.VectorSubcoreMesh(core_axis_name="c", subcore_axis_name="s")
    return functools.partial(
        pl.kernel,
        out_type=jax.ShapeDtypeStruct((_SEQ, _DB, batch // _W, 8, _W),
                                      jnp.float32),
        mesh=mesh,
        scratch_types=[
            pltpu.VMEM((_SEQ, _W), jnp.int32),
            pltpu.VMEM((2, _W, _PAD), jnp.float32),
            pltpu.VMEM((2, _DB, 8, _TSTRIDE), jnp.float32),
            pltpu.VMEM((_SEQ, _D), jnp.float32),
            pltpu.SemaphoreType.DMA,
            pltpu.SemaphoreType.DMA,
            pltpu.SemaphoreType.DMA,
            pltpu.SemaphoreType.DMA,
        ],
        compiler_params=pltpu.CompilerParams(
            use_tc_tiling_on_sc=False, needs_layout_passes=False),
    )(functools.partial(_emb_body, batch))


def kernel(input_ids, token_table, position_table):
    batch, seq = input_ids.shape
    ids_t = input_ids.astype(jnp.int32).T
    out5 = _make_lookup(batch)(ids_t, token_table, position_table)
    return out5.transpose(2, 4, 0, 1, 3).reshape(batch, seq, _D)


# speedup vs baseline: 1.3625x; 1.3625x over previous
import functools

import jax
import jax.numpy as jnp
from jax import lax
from jax.experimental import pallas as pl
from jax.experimental.pallas import tpu as pltpu
from jax.experimental.pallas import tpu_sc as plsc

_D = 64
_SEQ = 200
_LANES = 16
_PAD = 128

_NC = 2
_NS = 16
_NW = _NC * _NS

_W = 128
_DB = _D // 8
_STRIDE = _D + 1
_TSTRIDE = _W + 1


def _emb_body(batch, ids_hbm, tok_hbm, pos_hbm, out_hbm, idx_all, rows_v,
              tr_v, pos_v, gsem0, gsem1, ssem0, ssem1):
    wid = lax.axis_index("s") * _NC + lax.axis_index("c")
    b0 = wid * _W

    pltpu.sync_copy(ids_hbm.at[:, pl.ds(b0, _W)], idx_all)
    pltpu.sync_copy(pos_hbm, pos_v)

    lane = lax.iota(jnp.int32, _LANES)
    gsems = (gsem0, gsem1)
    ssems = (ssem0, ssem1)

    def fire_gather(s, q):
        pltpu.async_copy(tok_hbm.at[idx_all.at[s]], rows_v.at[q], gsems[q])

    def wait_gather(s, q):
        pltpu.make_async_copy(tok_hbm.at[idx_all.at[s]], rows_v.at[q],
                              gsems[q]).wait()

    def tr_view(p):
        return tr_v.at[p, :, :, pl.ds(0, _W)]

    def fire_store(s, p):
        pltpu.async_copy(tr_view(p), out_hbm.at[s, :, wid, :, :], ssems[p])

    def wait_store(s, p):
        pltpu.make_async_copy(tr_view(p), out_hbm.at[s, :, wid, :, :],
                              ssems[p]).wait()

    def stage(s, p):
        q = 1 - p

        @pl.when(s + 1 < _SEQ)
        def _():
            fire_gather(s + 1, q)

        wait_gather(s, p)

        @pl.when(s >= 2)
        def _():
            wait_store(s - 2, p)

        posk = [pos_v[s, pl.ds(k * _LANES, _LANES)]
                for k in range(_D // _LANES)]
        dbv, d8v = [], []
        for k in range(_D // _LANES):
            d = k * _LANES + lane
            dbv.append(d >> 3)
            d8v.append(d & 7)

        @plsc.parallel_loop(0, _W, unroll=8)
        def blk_body(b):
            bvec = jnp.full((_LANES,), b, jnp.int32)
            for k in range(_D // _LANES):
                v = rows_v[p, b, pl.ds(k * _LANES, _LANES)] + posk[k]
                plsc.store_scatter(tr_v.at[p], [dbv[k], d8v[k], bvec], v)

        fire_store(s, p)

    fire_gather(0, 0)

    def loop_body(i, _):
        stage(2 * i, 0)
        stage(2 * i + 1, 1)
        return 0

    lax.fori_loop(0, _SEQ // 2, loop_body, 0)

    wait_store(_SEQ - 2, 0)
    wait_store(_SEQ - 1, 1)


def _make_lookup(batch):
    mesh = plsc.VectorSubcoreMesh(core_axis_name="c", subcore_axis_name="s")
    return functools.partial(
        pl.kernel,
        out_type=jax.ShapeDtypeStruct((_SEQ, _DB, batch // _W, 8, _W),
                                      jnp.float32),
        mesh=mesh,
        scratch_types=[
            pltpu.VMEM((_SEQ, _W), jnp.int32),
            pltpu.VMEM((2, _W, _PAD), jnp.float32),
            pltpu.VMEM((2, _DB, 8, _TSTRIDE), jnp.float32),
            pltpu.VMEM((_SEQ, _D), jnp.float32),
            pltpu.SemaphoreType.DMA,
            pltpu.SemaphoreType.DMA,
            pltpu.SemaphoreType.DMA,
            pltpu.SemaphoreType.DMA,
        ],
        compiler_params=pltpu.CompilerParams(
            use_tc_tiling_on_sc=False, needs_layout_passes=False),
    )(functools.partial(_emb_body, batch))


def kernel(input_ids, token_table, position_table):
    batch, seq = input_ids.shape
    ids_t = input_ids.astype(jnp.int32).T
    tok_pad = jnp.pad(token_table, ((0, 0), (0, _PAD - _D)))
    out5 = _make_lookup(batch)(ids_t, tok_pad, position_table)
    return out5.transpose(2, 4, 0, 1, 3).reshape(batch, seq, _D)


# speedup vs baseline: 1.3983x; 1.0263x over previous
import functools

import jax
import jax.numpy as jnp
from jax import lax
from jax.experimental import pallas as pl
from jax.experimental.pallas import tpu as pltpu
from jax.experimental.pallas import tpu_sc as plsc

_D = 64
_SEQ = 200
_LANES = 16
_PAD = 128

_NC = 2
_NS = 16
_NW = _NC * _NS

_W = 128
_DB = _D // 8
_STRIDE = _D + 1
_TSTRIDE = _W + 1


def _emb_body(batch, ids_hbm, tok_hbm, pos_hbm, out_hbm, idx_all, rows_v,
              tr_v, pos_v, gsem0, gsem1, ssem0, ssem1):
    wid = lax.axis_index("s") * _NC + lax.axis_index("c")
    b0 = wid * _W

    pltpu.sync_copy(ids_hbm.at[:, pl.ds(b0, _W)], idx_all)
    pltpu.sync_copy(pos_hbm, pos_v)

    lane = lax.iota(jnp.int32, _LANES)
    gsems = (gsem0, gsem1)
    ssems = (ssem0, ssem1)

    def fire_gather(s, q):
        pltpu.async_copy(tok_hbm.at[idx_all.at[s]], rows_v.at[q], gsems[q])

    def wait_gather(s, q):
        pltpu.make_async_copy(tok_hbm.at[idx_all.at[s]], rows_v.at[q],
                              gsems[q]).wait()

    def tr_view(p):
        return tr_v.at[p, :, :, pl.ds(0, _W)]

    def fire_store(s, p):
        pltpu.async_copy(tr_view(p), out_hbm.at[s, :, wid, :, :], ssems[p])

    def wait_store(s, p):
        pltpu.make_async_copy(tr_view(p), out_hbm.at[s, :, wid, :, :],
                              ssems[p]).wait()

    def stage(s, p):
        q = 1 - p

        @pl.when(s + 1 < _SEQ)
        def _():
            fire_gather(s + 1, q)

        wait_gather(s, p)

        @pl.when(s >= 2)
        def _():
            wait_store(s - 2, p)

        posk = [pos_v[s, pl.ds(k * _LANES, _LANES)]
                for k in range(_D // _LANES)]
        dbv, d8v = [], []
        for k in range(_D // _LANES):
            d = k * _LANES + lane
            dbv.append(d >> 3)
            d8v.append(d & 7)

        @plsc.parallel_loop(0, _W, unroll=16)
        def blk_body(b):
            bvec = jnp.full((_LANES,), b, jnp.int32)
            for k in range(_D // _LANES):
                v = rows_v[p, b, pl.ds(k * _LANES, _LANES)] + posk[k]
                plsc.store_scatter(tr_v.at[p], [dbv[k], d8v[k], bvec], v)

        fire_store(s, p)

    fire_gather(0, 0)

    def loop_body(i, _):
        stage(2 * i, 0)
        stage(2 * i + 1, 1)
        return 0

    lax.fori_loop(0, _SEQ // 2, loop_body, 0)

    wait_store(_SEQ - 2, 0)
    wait_store(_SEQ - 1, 1)


def _make_lookup(batch):
    mesh = plsc.VectorSubcoreMesh(core_axis_name="c", subcore_axis_name="s")
    return functools.partial(
        pl.kernel,
        out_type=jax.ShapeDtypeStruct((_SEQ, _DB, batch // _W, 8, _W),
                                      jnp.float32),
        mesh=mesh,
        scratch_types=[
            pltpu.VMEM((_SEQ, _W), jnp.int32),
            pltpu.VMEM((2, _W, _PAD), jnp.float32),
            pltpu.VMEM((2, _DB, 8, _TSTRIDE), jnp.float32),
            pltpu.VMEM((_SEQ, _D), jnp.float32),
            pltpu.SemaphoreType.DMA,
            pltpu.SemaphoreType.DMA,
            pltpu.SemaphoreType.DMA,
            pltpu.SemaphoreType.DMA,
        ],
        compiler_params=pltpu.CompilerParams(
            use_tc_tiling_on_sc=False, needs_layout_passes=False),
    )(functools.partial(_emb_body, batch))


def kernel(input_ids, token_table, position_table):
    batch, seq = input_ids.shape
    ids_t = input_ids.astype(jnp.int32).T
    tok_pad = jnp.pad(token_table, ((0, 0), (0, _PAD - _D)))
    out5 = _make_lookup(batch)(ids_t, tok_pad, position_table)
    return out5.transpose(2, 4, 0, 1, 3).reshape(batch, seq, _D)
